# Initial kernel scaffold; baseline (speedup 1.0000x reference)
import jax
import jax.numpy as jnp
from jax.experimental import pallas as pl


def kernel(feature, bias, hate, table, W_bias, b_bias, W_hate, b_hate):
    raise NotImplementedError("write your pallas kernel here")


# speedup vs baseline: 12.5173x; 12.5173x over previous
import functools

import jax
import jax.numpy as jnp
from jax import lax
from jax.experimental import pallas as pl
from jax.experimental.pallas import tpu as pltpu
from jax.experimental.pallas import tpu_sc as plsc

NC, NS, LANES = 2, 16, 16
NW = NC * NS
B, SEQ = 4096, 200
BPW = B // NW
IPW = BPW * SEQ
C1, C2 = 128, 72
D = LANES


def _sc_body(feat_hbm, table_hbm, out_hbm, idx_v, buf_v, sout_v, sem):
    wid = lax.axis_index("s") * NC + lax.axis_index("c")
    base = pl.multiple_of(wid * IPW, 8)
    pltpu.sync_copy(feat_hbm.at[pl.ds(base, IPW)], idx_v)

    def row_body(b, _):
        off = pl.multiple_of(b * SEQ, 8)
        cp1 = pltpu.async_copy(
            table_hbm.at[idx_v.at[pl.ds(off, C1)]], buf_v.at[pl.ds(0, C1)], sem)
        cp2 = pltpu.async_copy(
            table_hbm.at[idx_v.at[pl.ds(off + C1, C2)]], buf_v.at[pl.ds(C1, C2)], sem)
        cp1.wait()
        cp2.wait()

        def acc_body(i, carry):
            a0, a1, a2, a3 = carry
            j = i * 4
            return (a0 + buf_v[j, :], a1 + buf_v[j + 1, :],
                    a2 + buf_v[j + 2, :], a3 + buf_v[j + 3, :])

        z = jnp.zeros((LANES,), jnp.float32)
        a0, a1, a2, a3 = lax.fori_loop(0, SEQ // 4, acc_body, (z, z, z, z),
                                       unroll=5)
        sout_v[b, :] = (a0 + a1) + (a2 + a3)
        return 0

    lax.fori_loop(0, BPW, row_body, 0)
    pltpu.sync_copy(sout_v, out_hbm.at[pl.ds(pl.multiple_of(wid * BPW, 8), BPW)])


@functools.lru_cache(maxsize=1)
def _sc_gather_sum():
    return pl.kernel(
        _sc_body,
        out_type=jax.ShapeDtypeStruct((B, D), jnp.float32),
        mesh=plsc.VectorSubcoreMesh(core_axis_name="c", subcore_axis_name="s",
                                    num_cores=NC, num_subcores=NS),
        compiler_params=pltpu.CompilerParams(use_tc_tiling_on_sc=False),
        scratch_types=[
            pltpu.VMEM((IPW,), jnp.int32),
            pltpu.VMEM((SEQ, D), jnp.float32),
            pltpu.VMEM((BPW, D), jnp.float32),
            pltpu.SemaphoreType.DMA,
        ],
    )


def _tc_body(s_ref, lab_ref, wb_ref, bb_ref, wh_ref, bh_ref,
             bl_ref, hl_ref, bp_ref, hp_ref):
    S = s_ref[...]
    labs = lab_ref[...]
    riota = lax.broadcasted_iota(jnp.int32, (4, B), 0)

    def head(W4, b4, lab_row):
        lt = lax.dot_general(W4, S, (((1,), (1,)), ((), ())),
                             precision=lax.Precision.HIGHEST,
                             preferred_element_type=jnp.float32)
        lt = lt + 200.0 * b4
        lt = jnp.where(riota < 3, lt, -jnp.inf)
        m = jnp.max(lt, axis=0, keepdims=True)
        lse = jnp.log(jnp.sum(jnp.exp(lt - m), axis=0, keepdims=True)) + m
        picked = jnp.sum(jnp.where(riota == lab_row, lt, 0.0), axis=0,
                         keepdims=True)
        loss = jnp.sum(lse - picked, axis=1, keepdims=True) / B
        l0, l1, l2 = lt[0:1], lt[1:2], lt[2:3]
        pred = jnp.where(l1 > l0, 1, 0)
        pred = jnp.where(l2 > jnp.maximum(l0, l1), 2, pred)
        return loss, pred.astype(jnp.int32)

    bl, bp = head(wb_ref[...], bb_ref[...], labs[0:1])
    hl, hp = head(wh_ref[...], bh_ref[...], labs[1:2])
    bl_ref[...] = bl
    hl_ref[...] = hl
    bp_ref[...] = bp
    hp_ref[...] = hp


_tc_heads = pl.pallas_call(
    _tc_body,
    out_shape=[
        jax.ShapeDtypeStruct((1, 1), jnp.float32),
        jax.ShapeDtypeStruct((1, 1), jnp.float32),
        jax.ShapeDtypeStruct((1, B), jnp.int32),
        jax.ShapeDtypeStruct((1, B), jnp.int32),
    ],
)


def kernel(feature, bias, hate, table, W_bias, b_bias, W_hate, b_hate):
    feat_flat = feature.astype(jnp.int32).reshape(-1)
    table_q = table.astype(jnp.bfloat16).astype(jnp.float32)
    w_bias_q = W_bias.astype(jnp.bfloat16).astype(jnp.float32)
    w_hate_q = W_hate.astype(jnp.bfloat16).astype(jnp.float32)
    table16 = jnp.pad(table_q, ((0, 0), (0, D - 3)))
    s16 = _sc_gather_sum()(feat_flat, table16)

    labs = jnp.stack([bias, hate]).astype(jnp.int32)
    w4b = jnp.zeros((4, D), jnp.float32).at[:3, :3].set(w_bias_q)
    b4b = jnp.zeros((4, 1), jnp.float32).at[:3, 0].set(b_bias)
    w4h = jnp.zeros((4, D), jnp.float32).at[:3, :3].set(w_hate_q)
    b4h = jnp.zeros((4, 1), jnp.float32).at[:3, 0].set(b_hate)

    bl, hl, bp, hp = _tc_heads(s16, labs, w4b, b4b, w4h, b4h)
    return (bl.reshape(()), hl.reshape(()), bp.reshape(B), hp.reshape(B))


# speedup vs baseline: 16.9089x; 1.3509x over previous
import functools

import jax
import jax.numpy as jnp
from jax import lax
from jax.experimental import pallas as pl
from jax.experimental.pallas import tpu as pltpu
from jax.experimental.pallas import tpu_sc as plsc

NC, NS, LANES = 2, 16, 16
NW = NC * NS
B, SEQ = 4096, 200
BPW = B // NW
IPW = BPW * SEQ
C1, C2 = 128, 72
D = LANES


def _sc_body(feat_hbm, table_hbm, out_hbm, idx_v, buf0_v, buf1_v, sout_v,
             sem0, sem1):
    wid = lax.axis_index("s") * NC + lax.axis_index("c")
    base = pl.multiple_of(wid * IPW, 8)
    pltpu.sync_copy(feat_hbm.at[pl.ds(base, IPW)], idx_v)

    def _mk(b, buf, sem):
        off = pl.multiple_of(b * SEQ, 8)
        c1 = pltpu.make_async_copy(
            table_hbm.at[idx_v.at[pl.ds(off, C1)]], buf.at[pl.ds(0, C1)], sem)
        c2 = pltpu.make_async_copy(
            table_hbm.at[idx_v.at[pl.ds(off + C1, C2)]], buf.at[pl.ds(C1, C2)],
            sem)
        return c1, c2

    def issue(b, buf, sem):
        c1, c2 = _mk(b, buf, sem)
        c1.start()
        c2.start()

    def drain_acc(b, buf, sem):
        c1, c2 = _mk(b, buf, sem)
        c1.wait()
        c2.wait()

        def acc_body(i, carry):
            a0, a1, a2, a3 = carry
            j = i * 4
            return (a0 + buf[j, :], a1 + buf[j + 1, :],
                    a2 + buf[j + 2, :], a3 + buf[j + 3, :])

        z = jnp.zeros((LANES,), jnp.float32)
        a0, a1, a2, a3 = lax.fori_loop(0, SEQ // 4, acc_body, (z, z, z, z),
                                       unroll=5)
        sout_v[b, :] = (a0 + a1) + (a2 + a3)

    issue(0, buf0_v, sem0)

    def g_body(g, _):
        b0 = g * 2
        issue(b0 + 1, buf1_v, sem1)
        drain_acc(b0, buf0_v, sem0)

        @pl.when(b0 + 2 < BPW)
        def _():
            issue(b0 + 2, buf0_v, sem0)

        drain_acc(b0 + 1, buf1_v, sem1)
        return 0

    lax.fori_loop(0, BPW // 2, g_body, 0)
    pltpu.sync_copy(sout_v, out_hbm.at[pl.ds(pl.multiple_of(wid * BPW, 8), BPW)])


@functools.lru_cache(maxsize=1)
def _sc_gather_sum():
    return pl.kernel(
        _sc_body,
        out_type=jax.ShapeDtypeStruct((B, D), jnp.float32),
        mesh=plsc.VectorSubcoreMesh(core_axis_name="c", subcore_axis_name="s",
                                    num_cores=NC, num_subcores=NS),
        compiler_params=pltpu.CompilerParams(use_tc_tiling_on_sc=False),
        scratch_types=[
            pltpu.VMEM((IPW,), jnp.int32),
            pltpu.VMEM((SEQ, D), jnp.float32),
            pltpu.VMEM((SEQ, D), jnp.float32),
            pltpu.VMEM((BPW, D), jnp.float32),
            pltpu.SemaphoreType.DMA,
            pltpu.SemaphoreType.DMA,
        ],
    )


def _tc_body(s_ref, lab_ref, wb_ref, bb_ref, wh_ref, bh_ref,
             bl_ref, hl_ref, bp_ref, hp_ref):
    S = s_ref[...]
    labs = lab_ref[...]
    riota = lax.broadcasted_iota(jnp.int32, (4, B), 0)

    def head(W4, b4, lab_row):
        lt = lax.dot_general(W4, S, (((1,), (1,)), ((), ())),
                             precision=lax.Precision.HIGHEST,
                             preferred_element_type=jnp.float32)
        lt = lt + 200.0 * b4
        lt = jnp.where(riota < 3, lt, -jnp.inf)
        m = jnp.max(lt, axis=0, keepdims=True)
        lse = jnp.log(jnp.sum(jnp.exp(lt - m), axis=0, keepdims=True)) + m
        picked = jnp.sum(jnp.where(riota == lab_row, lt, 0.0), axis=0,
                         keepdims=True)
        loss = jnp.sum(lse - picked, axis=1, keepdims=True) / B
        l0, l1, l2 = lt[0:1], lt[1:2], lt[2:3]
        pred = jnp.where(l1 > l0, 1, 0)
        pred = jnp.where(l2 > jnp.maximum(l0, l1), 2, pred)
        return loss, pred.astype(jnp.int32)

    bl, bp = head(wb_ref[...], bb_ref[...], labs[0:1])
    hl, hp = head(wh_ref[...], bh_ref[...], labs[1:2])
    bl_ref[...] = bl
    hl_ref[...] = hl
    bp_ref[...] = bp
    hp_ref[...] = hp


_tc_heads = pl.pallas_call(
    _tc_body,
    out_shape=[
        jax.ShapeDtypeStruct((1, 1), jnp.float32),
        jax.ShapeDtypeStruct((1, 1), jnp.float32),
        jax.ShapeDtypeStruct((1, B), jnp.int32),
        jax.ShapeDtypeStruct((1, B), jnp.int32),
    ],
)


def kernel(feature, bias, hate, table, W_bias, b_bias, W_hate, b_hate):
    feat_flat = feature.astype(jnp.int32).reshape(-1)
    table_q = table.astype(jnp.bfloat16).astype(jnp.float32)
    w_bias_q = W_bias.astype(jnp.bfloat16).astype(jnp.float32)
    w_hate_q = W_hate.astype(jnp.bfloat16).astype(jnp.float32)
    table16 = jnp.pad(table_q, ((0, 0), (0, D - 3)))
    s16 = _sc_gather_sum()(feat_flat, table16)

    labs = jnp.stack([bias, hate]).astype(jnp.int32)
    w4b = jnp.zeros((4, D), jnp.float32).at[:3, :3].set(w_bias_q)
    b4b = jnp.zeros((4, 1), jnp.float32).at[:3, 0].set(b_bias)
    w4h = jnp.zeros((4, D), jnp.float32).at[:3, :3].set(w_hate_q)
    b4h = jnp.zeros((4, 1), jnp.float32).at[:3, 0].set(b_hate)

    bl, hl, bp, hp = _tc_heads(s16, labs, w4b, b4b, w4h, b4h)
    return (bl.reshape(()), hl.reshape(()), bp.reshape(B), hp.reshape(B))


# speedup vs baseline: 20.2731x; 1.1990x over previous
import functools

import jax
import jax.numpy as jnp
from jax import lax
from jax.experimental import pallas as pl
from jax.experimental.pallas import tpu as pltpu
from jax.experimental.pallas import tpu_sc as plsc

NC, NS, LANES = 2, 16, 16
NW = NC * NS
B, SEQ = 4096, 200
BPW = B // NW
IPW = BPW * SEQ
GR = 8
GI = GR * SEQ
NG = BPW // GR
D = LANES


def _sc_body(feat_hbm, table_hbm, out_hbm, idx_v, buf0_v, buf1_v, sout_v,
             sem0, sem1):
    wid = lax.axis_index("s") * NC + lax.axis_index("c")
    base = pl.multiple_of(wid * IPW, 8)
    pltpu.sync_copy(feat_hbm.at[pl.ds(base, IPW)], idx_v)

    def _mk(g, buf, sem):
        off = pl.multiple_of(g * GI, 8)
        return pltpu.make_async_copy(
            table_hbm.at[idx_v.at[pl.ds(off, GI)]], buf, sem)

    def issue(g, buf, sem):
        _mk(g, buf, sem).start()

    def drain_acc(g, buf, sem):
        _mk(g, buf, sem).wait()
        for r in range(GR):
            def acc_body(i, carry):
                a0, a1, a2, a3 = carry
                j = r * SEQ + i * 4
                return (a0 + buf[j, :], a1 + buf[j + 1, :],
                        a2 + buf[j + 2, :], a3 + buf[j + 3, :])

            z = jnp.zeros((LANES,), jnp.float32)
            a0, a1, a2, a3 = lax.fori_loop(0, SEQ // 4, acc_body, (z, z, z, z),
                                           unroll=5)
            sout_v[g * GR + r, :] = (a0 + a1) + (a2 + a3)

    issue(0, buf0_v, sem0)

    def g_body(h, _):
        g0 = h * 2
        issue(g0 + 1, buf1_v, sem1)
        drain_acc(g0, buf0_v, sem0)

        @pl.when(g0 + 2 < NG)
        def _():
            issue(g0 + 2, buf0_v, sem0)

        drain_acc(g0 + 1, buf1_v, sem1)
        return 0

    lax.fori_loop(0, NG // 2, g_body, 0)
    pltpu.sync_copy(sout_v, out_hbm.at[pl.ds(pl.multiple_of(wid * BPW, 8), BPW)])


@functools.lru_cache(maxsize=1)
def _sc_gather_sum():
    return pl.kernel(
        _sc_body,
        out_type=jax.ShapeDtypeStruct((B, D), jnp.float32),
        mesh=plsc.VectorSubcoreMesh(core_axis_name="c", subcore_axis_name="s",
                                    num_cores=NC, num_subcores=NS),
        compiler_params=pltpu.CompilerParams(use_tc_tiling_on_sc=False),
        scratch_types=[
            pltpu.VMEM((IPW,), jnp.int32),
            pltpu.VMEM((GI, D), jnp.float32),
            pltpu.VMEM((GI, D), jnp.float32),
            pltpu.VMEM((BPW, D), jnp.float32),
            pltpu.SemaphoreType.DMA,
            pltpu.SemaphoreType.DMA,
        ],
    )


def _tc_body(s_ref, lab_ref, wb_ref, bb_ref, wh_ref, bh_ref,
             bl_ref, hl_ref, bp_ref, hp_ref):
    S = s_ref[...]
    labs = lab_ref[...]
    riota = lax.broadcasted_iota(jnp.int32, (4, B), 0)

    def head(W4, b4, lab_row):
        lt = lax.dot_general(W4, S, (((1,), (1,)), ((), ())),
                             precision=lax.Precision.HIGHEST,
                             preferred_element_type=jnp.float32)
        lt = lt + 200.0 * b4
        lt = jnp.where(riota < 3, lt, -jnp.inf)
        m = jnp.max(lt, axis=0, keepdims=True)
        lse = jnp.log(jnp.sum(jnp.exp(lt - m), axis=0, keepdims=True)) + m
        picked = jnp.sum(jnp.where(riota == lab_row, lt, 0.0), axis=0,
                         keepdims=True)
        loss = jnp.sum(lse - picked, axis=1, keepdims=True) / B
        l0, l1, l2 = lt[0:1], lt[1:2], lt[2:3]
        pred = jnp.where(l1 > l0, 1, 0)
        pred = jnp.where(l2 > jnp.maximum(l0, l1), 2, pred)
        return loss, pred.astype(jnp.int32)

    bl, bp = head(wb_ref[...], bb_ref[...], labs[0:1])
    hl, hp = head(wh_ref[...], bh_ref[...], labs[1:2])
    bl_ref[...] = bl
    hl_ref[...] = hl
    bp_ref[...] = bp
    hp_ref[...] = hp


_tc_heads = pl.pallas_call(
    _tc_body,
    out_shape=[
        jax.ShapeDtypeStruct((1, 1), jnp.float32),
        jax.ShapeDtypeStruct((1, 1), jnp.float32),
        jax.ShapeDtypeStruct((1, B), jnp.int32),
        jax.ShapeDtypeStruct((1, B), jnp.int32),
    ],
)


def kernel(feature, bias, hate, table, W_bias, b_bias, W_hate, b_hate):
    feat_flat = feature.astype(jnp.int32).reshape(-1)
    table_q = table.astype(jnp.bfloat16).astype(jnp.float32)
    w_bias_q = W_bias.astype(jnp.bfloat16).astype(jnp.float32)
    w_hate_q = W_hate.astype(jnp.bfloat16).astype(jnp.float32)
    table16 = jnp.pad(table_q, ((0, 0), (0, D - 3)))
    s16 = _sc_gather_sum()(feat_flat, table16)

    labs = jnp.stack([bias, hate]).astype(jnp.int32)
    w4b = jnp.zeros((4, D), jnp.float32).at[:3, :3].set(w_bias_q)
    b4b = jnp.zeros((4, 1), jnp.float32).at[:3, 0].set(b_bias)
    w4h = jnp.zeros((4, D), jnp.float32).at[:3, :3].set(w_hate_q)
    b4h = jnp.zeros((4, 1), jnp.float32).at[:3, 0].set(b_hate)

    bl, hl, bp, hp = _tc_heads(s16, labs, w4b, b4b, w4h, b4h)
    return (bl.reshape(()), hl.reshape(()), bp.reshape(B), hp.reshape(B))


# speedup vs baseline: 21.5650x; 1.0637x over previous
import functools

import jax
import jax.numpy as jnp
from jax import lax
from jax.experimental import pallas as pl
from jax.experimental.pallas import tpu as pltpu
from jax.experimental.pallas import tpu_sc as plsc

NC, NS, LANES = 2, 16, 16
NW = NC * NS
B, SEQ = 4096, 200
BPW = B // NW
IPW = BPW * SEQ
GR = 4
GI = GR * SEQ
NG = BPW // GR
D = LANES
VOCAB_P = 100096
VPS = VOCAB_P // NS


def _sc_body(feat_hbm, table_hbm, out_hbm, idx0_v, idx1_v, buf0_v, buf1_v,
             sout_v, stab_sh, semi0, semi1, semg0, semg1):
    wid = lax.axis_index("s") * NC + lax.axis_index("c")
    sid = lax.axis_index("s")
    base = pl.multiple_of(wid * IPW, 8)
    toff = pl.multiple_of(sid * VPS, 8)
    pltpu.sync_copy(table_hbm.at[pl.ds(toff, VPS)], stab_sh.at[pl.ds(toff, VPS)])

    def idx_cp(g, ib, sem):
        off = pl.multiple_of(base + g * GI, 8)
        return pltpu.make_async_copy(feat_hbm.at[pl.ds(off, GI)], ib, sem)

    def gat(ib, rb, sem):
        return pltpu.make_async_copy(stab_sh.at[ib], rb, sem)

    def acc(g, buf):
        for r in range(GR):
            def acc_body(i, carry):
                a0, a1, a2, a3 = carry
                j = r * SEQ + i * 4
                return (a0 + buf[j, :], a1 + buf[j + 1, :],
                        a2 + buf[j + 2, :], a3 + buf[j + 3, :])

            z = jnp.zeros((LANES,), jnp.float32)
            a0, a1, a2, a3 = lax.fori_loop(0, SEQ // 4, acc_body, (z, z, z, z),
                                           unroll=5)
            sout_v[g * GR + r, :] = (a0 + a1) + (a2 + a3)

    idx_cp(0, idx0_v, semi0).start()
    plsc.subcore_barrier()
    idx_cp(0, idx0_v, semi0).wait()
    gat(idx0_v, buf0_v, semg0).start()
    idx_cp(1, idx1_v, semi1).start()

    def g_body(h, _):
        g0 = h * 2
        idx_cp(g0 + 1, idx1_v, semi1).wait()
        gat(idx1_v, buf1_v, semg1).start()
        gat(idx0_v, buf0_v, semg0).wait()

        @pl.when(g0 + 2 < NG)
        def _():
            idx_cp(g0 + 2, idx0_v, semi0).start()

        acc(g0, buf0_v)

        @pl.when(g0 + 2 < NG)
        def _():
            idx_cp(g0 + 2, idx0_v, semi0).wait()
            gat(idx0_v, buf0_v, semg0).start()

        gat(idx1_v, buf1_v, semg1).wait()

        @pl.when(g0 + 3 < NG)
        def _():
            idx_cp(g0 + 3, idx1_v, semi1).start()

        acc(g0 + 1, buf1_v)
        return 0

    lax.fori_loop(0, NG // 2, g_body, 0)
    pltpu.sync_copy(sout_v, out_hbm.at[pl.ds(pl.multiple_of(wid * BPW, 8), BPW)])


@functools.lru_cache(maxsize=1)
def _sc_gather_sum():
    return pl.kernel(
        _sc_body,
        out_type=jax.ShapeDtypeStruct((B, D), jnp.float32),
        mesh=plsc.VectorSubcoreMesh(core_axis_name="c", subcore_axis_name="s",
                                    num_cores=NC, num_subcores=NS),
        compiler_params=pltpu.CompilerParams(use_tc_tiling_on_sc=False),
        scratch_types=[
            pltpu.VMEM((GI,), jnp.int32),
            pltpu.VMEM((GI,), jnp.int32),
            pltpu.VMEM((GI, D), jnp.float32),
            pltpu.VMEM((GI, D), jnp.float32),
            pltpu.VMEM((BPW, D), jnp.float32),
            pltpu.VMEM_SHARED((VOCAB_P, D), jnp.float32),
            pltpu.SemaphoreType.DMA,
            pltpu.SemaphoreType.DMA,
            pltpu.SemaphoreType.DMA,
            pltpu.SemaphoreType.DMA,
        ],
    )


def _tc_body(s_ref, lab_ref, wb_ref, bb_ref, wh_ref, bh_ref,
             bl_ref, hl_ref, bp_ref, hp_ref):
    S = s_ref[...]
    labs = lab_ref[...]
    riota = lax.broadcasted_iota(jnp.int32, (4, B), 0)

    def head(W4, b4, lab_row):
        lt = lax.dot_general(W4, S, (((1,), (1,)), ((), ())),
                             precision=lax.Precision.HIGHEST,
                             preferred_element_type=jnp.float32)
        lt = lt + 200.0 * b4
        lt = jnp.where(riota < 3, lt, -jnp.inf)
        m = jnp.max(lt, axis=0, keepdims=True)
        lse = jnp.log(jnp.sum(jnp.exp(lt - m), axis=0, keepdims=True)) + m
        picked = jnp.sum(jnp.where(riota == lab_row, lt, 0.0), axis=0,
                         keepdims=True)
        loss = jnp.sum(lse - picked, axis=1, keepdims=True) / B
        l0, l1, l2 = lt[0:1], lt[1:2], lt[2:3]
        pred = jnp.where(l1 > l0, 1, 0)
        pred = jnp.where(l2 > jnp.maximum(l0, l1), 2, pred)
        return loss, pred.astype(jnp.int32)

    bl, bp = head(wb_ref[...], bb_ref[...], labs[0:1])
    hl, hp = head(wh_ref[...], bh_ref[...], labs[1:2])
    bl_ref[...] = bl
    hl_ref[...] = hl
    bp_ref[...] = bp
    hp_ref[...] = hp


_tc_heads = pl.pallas_call(
    _tc_body,
    out_shape=[
        jax.ShapeDtypeStruct((1, 1), jnp.float32),
        jax.ShapeDtypeStruct((1, 1), jnp.float32),
        jax.ShapeDtypeStruct((1, B), jnp.int32),
        jax.ShapeDtypeStruct((1, B), jnp.int32),
    ],
)


def kernel(feature, bias, hate, table, W_bias, b_bias, W_hate, b_hate):
    feat_flat = feature.astype(jnp.int32).reshape(-1)
    table_q = table.astype(jnp.bfloat16).astype(jnp.float32)
    w_bias_q = W_bias.astype(jnp.bfloat16).astype(jnp.float32)
    w_hate_q = W_hate.astype(jnp.bfloat16).astype(jnp.float32)
    table16 = jnp.pad(table_q, ((0, VOCAB_P - table.shape[0]), (0, D - 3)))
    s16 = _sc_gather_sum()(feat_flat, table16)

    labs = jnp.stack([bias, hate]).astype(jnp.int32)
    w4b = jnp.zeros((4, D), jnp.float32).at[:3, :3].set(w_bias_q)
    b4b = jnp.zeros((4, 1), jnp.float32).at[:3, 0].set(b_bias)
    w4h = jnp.zeros((4, D), jnp.float32).at[:3, :3].set(w_hate_q)
    b4h = jnp.zeros((4, 1), jnp.float32).at[:3, 0].set(b_hate)

    bl, hl, bp, hp = _tc_heads(s16, labs, w4b, b4b, w4h, b4h)
    return (bl.reshape(()), hl.reshape(()), bp.reshape(B), hp.reshape(B))
